# baseline (speedup 1.0000x reference)
import jax
import jax.numpy as jnp
import numpy as np
from jax.experimental import pallas as pl
from jax.experimental.pallas import tpu as pltpu

_VERT_IDX = np.array([1382, 1522, 1541, 1667, 1493, 428, 179, 1806, 2289,
                      2408, 2405, 2442, 19, 2504, 3016, 3164, 3049, 3060,
                      364, 626, 3454, 3756, 3863, 3844, 3915, 0, 0, 0, 0,
                      0, 0, 0, 2420, 2332, 2131, 2241, 3129, 3133, 2895,
                      3005, 3815, 3778, 3644, 3713, 0, 0], dtype=np.int64)

_B = 4096
_V = 4040
_K = _VERT_IDX.shape[0]


def _gather_body(vt_ref, out_ref, sem):
    def copy(c, k):
        return pltpu.make_async_copy(
            vt_ref.at[pl.ds(c, 1), pl.ds(int(_VERT_IDX[k]), 1), :],
            out_ref.at[pl.ds(c, 1), pl.ds(k, 1), :],
            sem,
        )

    for c in range(3):
        for k in range(_K):
            copy(c, k).start()
    for c in range(3):
        for k in range(_K):
            copy(c, k).wait()


def kernel(vertices):
    vt = jnp.transpose(vertices, (2, 1, 0))
    out_t = pl.pallas_call(
        _gather_body,
        in_specs=[pl.BlockSpec(memory_space=pltpu.MemorySpace.HBM)],
        out_specs=pl.BlockSpec(memory_space=pltpu.MemorySpace.HBM),
        out_shape=jax.ShapeDtypeStruct((3, _K, _B), jnp.float32),
        scratch_shapes=[pltpu.SemaphoreType.DMA],
    )(vt)
    return jnp.transpose(out_t, (2, 1, 0))


# speedup vs baseline: 11.3082x; 11.3082x over previous
import jax
import jax.numpy as jnp
import numpy as np
from jax.experimental import pallas as pl
from jax.experimental.pallas import tpu as pltpu

_VERT_IDX = np.array([1382, 1522, 1541, 1667, 1493, 428, 179, 1806, 2289,
                      2408, 2405, 2442, 19, 2504, 3016, 3164, 3049, 3060,
                      364, 626, 3454, 3756, 3863, 3844, 3915, 0, 0, 0, 0,
                      0, 0, 0, 2420, 2332, 2131, 2241, 3129, 3133, 2895,
                      3005, 3815, 3778, 3644, 3713, 0, 0], dtype=np.int64)

_B = 4096
_V = 4040
_K = _VERT_IDX.shape[0]


def _gather_body(vt_ref, out_ref, sem):
    c = pl.program_id(0)

    def copy(k):
        return pltpu.make_async_copy(
            vt_ref.at[pl.ds(c, 1), pl.ds(int(_VERT_IDX[k]), 1), :],
            out_ref.at[:, pl.ds(k, 1), :],
            sem,
        )

    for k in range(_K):
        copy(k).start()
    for k in range(_K):
        copy(k).wait()


def kernel(vertices):
    vt = jnp.transpose(vertices, (2, 1, 0))
    out_t = pl.pallas_call(
        _gather_body,
        grid=(3,),
        in_specs=[pl.BlockSpec(memory_space=pltpu.MemorySpace.HBM)],
        out_specs=pl.BlockSpec((1, _K, _B), lambda c: (c, 0, 0)),
        out_shape=jax.ShapeDtypeStruct((3, _K, _B), jnp.float32),
        scratch_shapes=[pltpu.SemaphoreType.DMA],
    )(vt)
    return jnp.transpose(out_t, (2, 1, 0))
